# baseline (speedup 1.0000x reference)
import functools

import jax
import jax.numpy as jnp
import numpy as np
from jax import lax
from jax.experimental import pallas as pl
from jax.experimental.pallas import tpu as pltpu
from jax.experimental.pallas import tpu_sc as plsc

N_POS = 262144
FEATS = 4
T3 = 8192
T2 = 32768
RES3_LIST = [int(16 * (512.0 / 16.0) ** (i / 11.0)) for i in range(12)]
RES2_LIST = [int(128 * (1024.0 / 128.0) ** (i / 3.0)) for i in range(4)]
NLEV3 = len(RES3_LIST)
NLEV2 = len(RES2_LIST)
NFEAT_OUT = NLEV3 * FEATS + NLEV2 * FEATS

K2 = int(np.uint32(2654435761).astype(np.int32))
K3 = 805459861
SIGN = -2147483648

W3_PER_LEV = T3 * FEATS // 32
W2_PER_LEV = T2 * FEATS // 32
W3_TOTAL = NLEV3 * W3_PER_LEV
NWORDS = W3_TOTAL + 3 * NLEV2 * W2_PER_LEV

NC, NS, L = 2, 16, 16
NWK = NC * NS
P_PER_WK = N_POS // NWK
CHUNK = 256
NG = CHUNK // L
NCHUNK = P_PER_WK // CHUNK

_PACK_BLK_ROWS = 4096
_WPB = _PACK_BLK_ROWS // 8


def _pack_body(x_ref, o_ref):
    x = x_ref[...].reshape(_WPB, 8, FEATS)
    r = lax.broadcasted_iota(jnp.int32, (_WPB, 8, FEATS), 1)
    f = lax.broadcasted_iota(jnp.int32, (_WPB, 8, FEATS), 2)
    sv = lax.shift_left(jnp.int32(1), r * FEATS + f)
    vals = jnp.where(x < 0, sv, jnp.int32(0))
    o_ref[...] = jnp.sum(vals, axis=(1, 2)).reshape(1, 1, _WPB)


def _pack_rows(rows4):
    nblk = rows4.shape[0] // _PACK_BLK_ROWS
    out = pl.pallas_call(
        _pack_body,
        grid=(nblk,),
        in_specs=[pl.BlockSpec((_PACK_BLK_ROWS, FEATS), lambda i: (i, 0))],
        out_specs=pl.BlockSpec((1, 1, _WPB), lambda i: (i, 0, 0)),
        out_shape=jax.ShapeDtypeStruct((nblk, 1, _WPB), jnp.int32),
    )(rows4)
    return out.reshape(nblk * _WPB)


def _corner_accum(tab_v, hraw, wgt, base, tmask, acc):
    hm = hraw & tmask
    word = lax.shift_right_logical(hm, 3) + base
    s0 = 28 - lax.shift_left(hm & 7, 2)
    pk = plsc.load_gather(tab_v, [word])
    t0 = lax.shift_left(pk, s0)
    wb = lax.bitcast_convert_type(wgt, jnp.int32)
    out = []
    for f in range(FEATS):
        tf = t0 if f == 3 else lax.shift_left(t0, 3 - f)
        sb = tf & SIGN
        v = lax.bitcast_convert_type(sb ^ wb, jnp.float32)
        out.append(v if acc is None else acc[f] + v)
    return out


def _make_sc_kernel():
    mesh = plsc.VectorSubcoreMesh(core_axis_name="c", subcore_axis_name="s")

    @functools.partial(
        pl.kernel,
        mesh=mesh,
        out_type=jax.ShapeDtypeStruct((N_POS, NFEAT_OUT), jnp.float32),
        compiler_params=pltpu.CompilerParams(needs_layout_passes=False),
        scratch_types=[
            pltpu.VMEM((NWORDS,), jnp.int32),
            pltpu.VMEM((CHUNK,), jnp.float32),
            pltpu.VMEM((CHUNK,), jnp.float32),
            pltpu.VMEM((CHUNK,), jnp.float32),
            pltpu.VMEM((CHUNK,), jnp.float32),
            pltpu.VMEM((CHUNK,), jnp.float32),
            pltpu.VMEM((CHUNK,), jnp.float32),
            pltpu.VMEM((CHUNK, NFEAT_OUT), jnp.float32),
            pltpu.VMEM((CHUNK, NFEAT_OUT), jnp.float32),
            pltpu.VMEM((NLEV3 * L,), jnp.float32),
            pltpu.VMEM((NLEV2 * L,), jnp.float32),
            pltpu.SemaphoreType.DMA,
            pltpu.SemaphoreType.DMA,
            pltpu.SemaphoreType.DMA,
            pltpu.SemaphoreType.DMA,
        ],
    )
    def sc_kernel(packed_hbm, xs_hbm, ys_hbm, zs_hbm, r3_hbm, r2_hbm, out_hbm,
                  tab_v, xba, yba, zba, xbb, ybb, zbb, oba, obb, r3v, r2v,
                  sia, sib, soa, sob):
        wid = lax.axis_index("s") * NC + lax.axis_index("c")
        pltpu.sync_copy(packed_hbm, tab_v)
        pltpu.sync_copy(r3_hbm, r3v)
        pltpu.sync_copy(r2_hbm, r2v)
        iota = lax.broadcasted_iota(jnp.int32, (L,), 0)
        izero = iota * 0
        base0 = wid * P_PER_WK
        bufs = ((xba, yba, zba, oba, sia, soa),
                (xbb, ybb, zbb, obb, sib, sob))

        def in_copies(ch, b):
            xb, yb, zb = bufs[b][0], bufs[b][1], bufs[b][2]
            base = base0 + ch * CHUNK
            return (
                pltpu.make_async_copy(xs_hbm.at[pl.ds(base, CHUNK)], xb,
                                      bufs[b][4]),
                pltpu.make_async_copy(ys_hbm.at[pl.ds(base, CHUNK)], yb,
                                      bufs[b][4]),
                pltpu.make_async_copy(zs_hbm.at[pl.ds(base, CHUNK)], zb,
                                      bufs[b][4]),
            )

        def out_copy(ch, b):
            base = base0 + ch * CHUNK
            return pltpu.make_async_copy(bufs[b][3],
                                         out_hbm.at[pl.ds(base, CHUNK)],
                                         bufs[b][5])

        def compute_chunk(xb, yb, zb, ob):
            @plsc.parallel_loop(0, NG)
            def group_body(g):
                xv = xb[pl.ds(g * L, L)]
                yv = yb[pl.ds(g * L, L)]
                zv = zb[pl.ds(g * L, L)]
                xn = jnp.minimum(jnp.maximum(xv, 0.0), 0.9999)
                yn = jnp.minimum(jnp.maximum(yv, 0.0), 0.9999)
                zn = jnp.minimum(jnp.maximum(zv, 0.0), 0.9999)
                rows = iota + g * L

                @plsc.parallel_loop(0, NLEV3, unroll=2)
                def lev3_body(lev):
                    res = r3v[pl.ds(lev * L, L)]
                    sx = xn * res
                    sy = yn * res
                    sz = zn * res
                    fx = sx.astype(jnp.int32)
                    fy = sy.astype(jnp.int32)
                    fz = sz.astype(jnp.int32)
                    wx = sx - fx.astype(jnp.float32)
                    wy = sy - fy.astype(jnp.float32)
                    wz = sz - fz.astype(jnp.float32)
                    ux = 1.0 - wx
                    uy = 1.0 - wy
                    uz = 1.0 - wz
                    by0 = fy * K2
                    by1 = by0 + K2
                    cz0 = fz * K3
                    cz1 = cz0 + K3
                    ax1 = fx + 1
                    xy00 = fx ^ by0
                    xy01 = fx ^ by1
                    xy10 = ax1 ^ by0
                    xy11 = ax1 ^ by1
                    w00 = ux * uy
                    w01 = ux * wy
                    w10 = wx * uy
                    w11 = wx * wy
                    wbase = lev * W3_PER_LEV
                    acc = None
                    for hxy, wxy in ((xy00, w00), (xy01, w01),
                                     (xy10, w10), (xy11, w11)):
                        acc = _corner_accum(tab_v, hxy ^ cz0, wxy * uz,
                                            wbase, T3 - 1, acc)
                        acc = _corner_accum(tab_v, hxy ^ cz1, wxy * wz,
                                            wbase, T3 - 1, acc)
                    col0 = izero + lev * FEATS
                    for f in range(FEATS):
                        plsc.store_scatter(ob, [rows, col0 + f], acc[f])

                @plsc.parallel_loop(0, NLEV2, unroll=2)
                def lev2_body(lev):
                    res = r2v[pl.ds(lev * L, L)]
                    wb2 = W3_TOTAL + lev * W2_PER_LEV
                    acc = None
                    for pi, (pc, qc) in enumerate(((xn, yn), (xn, zn), (yn, zn))):
                        su = pc * res
                        sv = qc * res
                        fu = su.astype(jnp.int32)
                        fv = sv.astype(jnp.int32)
                        wu = su - fu.astype(jnp.float32)
                        wv = sv - fv.astype(jnp.float32)
                        pu = 1.0 - wu
                        pv = 1.0 - wv
                        bv0 = fv * K2
                        bv1 = bv0 + K2
                        au1 = fu + 1
                        pbase = wb2 + pi * (NLEV2 * W2_PER_LEV)
                        acc = _corner_accum(tab_v, fu ^ bv0, pu * pv,
                                            pbase, T2 - 1, acc)
                        acc = _corner_accum(tab_v, fu ^ bv1, pu * wv,
                                            pbase, T2 - 1, acc)
                        acc = _corner_accum(tab_v, au1 ^ bv0, wu * pv,
                                            pbase, T2 - 1, acc)
                        acc = _corner_accum(tab_v, au1 ^ bv1, wu * wv,
                                            pbase, T2 - 1, acc)
                    col0 = izero + (NLEV3 * FEATS + lev * FEATS)
                    for f in range(FEATS):
                        plsc.store_scatter(ob, [rows, col0 + f], acc[f])

        for c in in_copies(0, 0):
            c.start()

        def chunk2_body(ch2, carry):
            for b in range(2):
                ch = ch2 * 2 + b
                if b == 0:
                    for c in in_copies(ch + 1, 1):
                        c.start()
                else:
                    @pl.when(ch2 < NCHUNK // 2 - 1)
                    def _():
                        for c in in_copies(ch + 1, 0):
                            c.start()
                for c in in_copies(ch, b):
                    c.wait()
                @pl.when(ch2 > 0)
                def _():
                    out_copy(ch, b).wait()
                compute_chunk(bufs[b][0], bufs[b][1], bufs[b][2], bufs[b][3])
                out_copy(ch, b).start()
            return carry

        lax.fori_loop(0, NCHUNK // 2, chunk2_body, 0)
        out_copy(NCHUNK - 2, 0).wait()
        out_copy(NCHUNK - 1, 1).wait()

    return sc_kernel


_SC_KERNEL = _make_sc_kernel()

_R3_ARR = np.repeat(np.asarray(RES3_LIST, np.float32), L)
_R2_ARR = np.repeat(np.asarray(RES2_LIST, np.float32), L)


@jax.jit
def kernel(positions, tables_3d, tables_2d_xy, tables_2d_xz, tables_2d_yz):
    packed = jnp.concatenate([
        _pack_rows(tables_3d.reshape(-1, FEATS)),
        _pack_rows(tables_2d_xy.reshape(-1, FEATS)),
        _pack_rows(tables_2d_xz.reshape(-1, FEATS)),
        _pack_rows(tables_2d_yz.reshape(-1, FEATS)),
    ])
    posT = positions.T
    return _SC_KERNEL(packed, posT[0], posT[1], posT[2],
                      jnp.asarray(_R3_ARR), jnp.asarray(_R2_ARR))


# speedup vs baseline: 1.0179x; 1.0179x over previous
import functools

import jax
import jax.numpy as jnp
import numpy as np
from jax import lax
from jax.experimental import pallas as pl
from jax.experimental.pallas import tpu as pltpu
from jax.experimental.pallas import tpu_sc as plsc

N_POS = 262144
FEATS = 4
T3 = 8192
T2 = 32768
RES3_LIST = [int(16 * (512.0 / 16.0) ** (i / 11.0)) for i in range(12)]
RES2_LIST = [int(128 * (1024.0 / 128.0) ** (i / 3.0)) for i in range(4)]
NLEV3 = len(RES3_LIST)
NLEV2 = len(RES2_LIST)
NFEAT_OUT = NLEV3 * FEATS + NLEV2 * FEATS

K2 = int(np.uint32(2654435761).astype(np.int32))
K3 = 805459861
SIGN = -2147483648

W3_PER_LEV = T3 * FEATS // 32
W2_PER_LEV = T2 * FEATS // 32
W3_TOTAL = NLEV3 * W3_PER_LEV
NWORDS = W3_TOTAL + 3 * NLEV2 * W2_PER_LEV

NC, NS, L = 2, 16, 16
NWK = NC * NS
P_PER_WK = N_POS // NWK
CHUNK = 256
NG = CHUNK // L
NCHUNK = P_PER_WK // CHUNK

_PACK_BLK_ROWS = 4096
_WPB = _PACK_BLK_ROWS // 8


def _pack_body(x_ref, o_ref):
    x = x_ref[...].reshape(_WPB, 8, FEATS)
    r = lax.broadcasted_iota(jnp.int32, (_WPB, 8, FEATS), 1)
    f = lax.broadcasted_iota(jnp.int32, (_WPB, 8, FEATS), 2)
    sv = lax.shift_left(jnp.int32(1), r * FEATS + f)
    vals = jnp.where(x < 0, sv, jnp.int32(0))
    o_ref[...] = jnp.sum(vals, axis=(1, 2)).reshape(1, 1, _WPB)


def _pack_rows(rows4):
    nblk = rows4.shape[0] // _PACK_BLK_ROWS
    out = pl.pallas_call(
        _pack_body,
        grid=(nblk,),
        in_specs=[pl.BlockSpec((_PACK_BLK_ROWS, FEATS), lambda i: (i, 0))],
        out_specs=pl.BlockSpec((1, 1, _WPB), lambda i: (i, 0, 0)),
        out_shape=jax.ShapeDtypeStruct((nblk, 1, _WPB), jnp.int32),
    )(rows4)
    return out.reshape(nblk * _WPB)


def _corner_accum(tab_ref, hraw, wgt, tmask, acc):
    hm = hraw & tmask
    word = lax.shift_right_logical(hm, 3)
    s0 = 28 - lax.shift_left(hm & 7, 2)
    pk = plsc.load_gather(tab_ref, [word])
    t0 = lax.shift_left(pk, s0)
    wb = lax.bitcast_convert_type(wgt, jnp.int32)
    out = []
    for f in range(FEATS):
        tf = t0 if f == 3 else lax.shift_left(t0, 3 - f)
        sb = tf & SIGN
        v = lax.bitcast_convert_type(sb ^ wb, jnp.float32)
        out.append(v if acc is None else acc[f] + v)
    return out


def _make_sc_kernel():
    mesh = plsc.VectorSubcoreMesh(core_axis_name="c", subcore_axis_name="s")

    @functools.partial(
        pl.kernel,
        mesh=mesh,
        out_type=jax.ShapeDtypeStruct((N_POS, NFEAT_OUT), jnp.float32),
        compiler_params=pltpu.CompilerParams(needs_layout_passes=False),
        scratch_types=[
            pltpu.VMEM((NWORDS,), jnp.int32),
            pltpu.VMEM((CHUNK,), jnp.float32),
            pltpu.VMEM((CHUNK,), jnp.float32),
            pltpu.VMEM((CHUNK,), jnp.float32),
            pltpu.VMEM((CHUNK,), jnp.float32),
            pltpu.VMEM((CHUNK,), jnp.float32),
            pltpu.VMEM((CHUNK,), jnp.float32),
            pltpu.VMEM((CHUNK, NFEAT_OUT), jnp.float32),
            pltpu.VMEM((CHUNK, NFEAT_OUT), jnp.float32),
            pltpu.VMEM((NLEV3 * L,), jnp.float32),
            pltpu.VMEM((NLEV2 * L,), jnp.float32),
            pltpu.SemaphoreType.DMA,
            pltpu.SemaphoreType.DMA,
            pltpu.SemaphoreType.DMA,
            pltpu.SemaphoreType.DMA,
        ],
    )
    def sc_kernel(packed_hbm, xs_hbm, ys_hbm, zs_hbm, r3_hbm, r2_hbm, out_hbm,
                  tab_v, xba, yba, zba, xbb, ybb, zbb, oba, obb, r3v, r2v,
                  sia, sib, soa, sob):
        wid = lax.axis_index("s") * NC + lax.axis_index("c")
        pltpu.sync_copy(packed_hbm, tab_v)
        pltpu.sync_copy(r3_hbm, r3v)
        pltpu.sync_copy(r2_hbm, r2v)
        iota = lax.broadcasted_iota(jnp.int32, (L,), 0)
        izero = iota * 0
        base0 = wid * P_PER_WK
        bufs = ((xba, yba, zba, oba, sia, soa),
                (xbb, ybb, zbb, obb, sib, sob))

        def in_copies(ch, b):
            xb, yb, zb = bufs[b][0], bufs[b][1], bufs[b][2]
            base = base0 + ch * CHUNK
            return (
                pltpu.make_async_copy(xs_hbm.at[pl.ds(base, CHUNK)], xb,
                                      bufs[b][4]),
                pltpu.make_async_copy(ys_hbm.at[pl.ds(base, CHUNK)], yb,
                                      bufs[b][4]),
                pltpu.make_async_copy(zs_hbm.at[pl.ds(base, CHUNK)], zb,
                                      bufs[b][4]),
            )

        def out_copy(ch, b):
            base = base0 + ch * CHUNK
            return pltpu.make_async_copy(bufs[b][3],
                                         out_hbm.at[pl.ds(base, CHUNK)],
                                         bufs[b][5])

        def compute_chunk(xb, yb, zb, ob):
            @plsc.parallel_loop(0, NG)
            def group_body(g):
                xv = xb[pl.ds(g * L, L)]
                yv = yb[pl.ds(g * L, L)]
                zv = zb[pl.ds(g * L, L)]
                xn = jnp.minimum(jnp.maximum(xv, 0.0), 0.9999)
                yn = jnp.minimum(jnp.maximum(yv, 0.0), 0.9999)
                zn = jnp.minimum(jnp.maximum(zv, 0.0), 0.9999)
                rows = iota + g * L

                @plsc.parallel_loop(0, NLEV3)
                def lev3_body(lev):
                    res = r3v[pl.ds(lev * L, L)]
                    sx = xn * res
                    sy = yn * res
                    sz = zn * res
                    fx = sx.astype(jnp.int32)
                    fy = sy.astype(jnp.int32)
                    fz = sz.astype(jnp.int32)
                    wx = sx - fx.astype(jnp.float32)
                    wy = sy - fy.astype(jnp.float32)
                    wz = sz - fz.astype(jnp.float32)
                    ux = 1.0 - wx
                    uy = 1.0 - wy
                    uz = 1.0 - wz
                    by0 = fy * K2
                    by1 = by0 + K2
                    cz0 = fz * K3
                    cz1 = cz0 + K3
                    ax1 = fx + 1
                    xy00 = fx ^ by0
                    xy01 = fx ^ by1
                    xy10 = ax1 ^ by0
                    xy11 = ax1 ^ by1
                    w00 = ux * uy
                    w01 = ux * wy
                    w10 = wx * uy
                    w11 = wx * wy
                    tab3 = tab_v.at[pl.ds(lev * W3_PER_LEV, W3_PER_LEV)]
                    acc = None
                    for hxy, wxy in ((xy00, w00), (xy01, w01),
                                     (xy10, w10), (xy11, w11)):
                        acc = _corner_accum(tab3, hxy ^ cz0, wxy * uz,
                                            T3 - 1, acc)
                        acc = _corner_accum(tab3, hxy ^ cz1, wxy * wz,
                                            T3 - 1, acc)
                    col0 = izero + lev * FEATS
                    for f in range(FEATS):
                        plsc.store_scatter(ob, [rows, col0 + f], acc[f])

                @plsc.parallel_loop(0, NLEV2)
                def lev2_body(lev):
                    res = r2v[pl.ds(lev * L, L)]
                    wb2 = W3_TOTAL + lev * W2_PER_LEV
                    acc = None
                    for pi, (pc, qc) in enumerate(((xn, yn), (xn, zn), (yn, zn))):
                        su = pc * res
                        sv = qc * res
                        fu = su.astype(jnp.int32)
                        fv = sv.astype(jnp.int32)
                        wu = su - fu.astype(jnp.float32)
                        wv = sv - fv.astype(jnp.float32)
                        pu = 1.0 - wu
                        pv = 1.0 - wv
                        bv0 = fv * K2
                        bv1 = bv0 + K2
                        au1 = fu + 1
                        tabp = tab_v.at[pl.ds(wb2 + pi * (NLEV2 * W2_PER_LEV),
                                              W2_PER_LEV)]
                        acc = _corner_accum(tabp, fu ^ bv0, pu * pv,
                                            T2 - 1, acc)
                        acc = _corner_accum(tabp, fu ^ bv1, pu * wv,
                                            T2 - 1, acc)
                        acc = _corner_accum(tabp, au1 ^ bv0, wu * pv,
                                            T2 - 1, acc)
                        acc = _corner_accum(tabp, au1 ^ bv1, wu * wv,
                                            T2 - 1, acc)
                    col0 = izero + (NLEV3 * FEATS + lev * FEATS)
                    for f in range(FEATS):
                        plsc.store_scatter(ob, [rows, col0 + f], acc[f])

        for c in in_copies(0, 0):
            c.start()

        def chunk2_body(ch2, carry):
            for b in range(2):
                ch = ch2 * 2 + b
                if b == 0:
                    for c in in_copies(ch + 1, 1):
                        c.start()
                else:
                    @pl.when(ch2 < NCHUNK // 2 - 1)
                    def _():
                        for c in in_copies(ch + 1, 0):
                            c.start()
                for c in in_copies(ch, b):
                    c.wait()
                @pl.when(ch2 > 0)
                def _():
                    out_copy(ch, b).wait()
                compute_chunk(bufs[b][0], bufs[b][1], bufs[b][2], bufs[b][3])
                out_copy(ch, b).start()
            return carry

        lax.fori_loop(0, NCHUNK // 2, chunk2_body, 0)
        out_copy(NCHUNK - 2, 0).wait()
        out_copy(NCHUNK - 1, 1).wait()

    return sc_kernel


_SC_KERNEL = _make_sc_kernel()

_R3_ARR = np.repeat(np.asarray(RES3_LIST, np.float32), L)
_R2_ARR = np.repeat(np.asarray(RES2_LIST, np.float32), L)


@jax.jit
def kernel(positions, tables_3d, tables_2d_xy, tables_2d_xz, tables_2d_yz):
    packed = jnp.concatenate([
        _pack_rows(tables_3d.reshape(-1, FEATS)),
        _pack_rows(tables_2d_xy.reshape(-1, FEATS)),
        _pack_rows(tables_2d_xz.reshape(-1, FEATS)),
        _pack_rows(tables_2d_yz.reshape(-1, FEATS)),
    ])
    posT = positions.T
    return _SC_KERNEL(packed, posT[0], posT[1], posT[2],
                      jnp.asarray(_R3_ARR), jnp.asarray(_R2_ARR))
